# Initial kernel scaffold; baseline (speedup 1.0000x reference)
import jax
import jax.numpy as jnp
from jax.experimental import pallas as pl


def kernel(k, src_ind, feat, W_theta, W_phi):
    raise NotImplementedError("write your pallas kernel here")


# speedup vs baseline: 1.5381x; 1.5381x over previous
import functools

import jax
import jax.numpy as jnp
from jax import lax
from jax.experimental import pallas as pl
from jax.experimental.pallas import tpu as pltpu
from jax.experimental.pallas import tpu_sc as plsc

D = 128
K = 32
NC = 2
NS = 16
NW = NC * NS
L = 16
B = 4


def _matmul_body(x_ref, wt_ref, wd_ref, hs_ref, hd_ref):
    x = x_ref[...]
    hs_ref[...] = jnp.dot(x, wt_ref[...], preferred_element_type=jnp.float32)
    hd_ref[...] = jnp.dot(x, wd_ref[...], preferred_element_type=jnp.float32)


def _tc_matmuls(feat_pad, wt, wd, n_pad):
    bm = 512
    grid = (n_pad // bm,)
    return pl.pallas_call(
        _matmul_body,
        grid=grid,
        in_specs=[
            pl.BlockSpec((bm, D), lambda i: (i, 0)),
            pl.BlockSpec((D, D), lambda i: (0, 0)),
            pl.BlockSpec((D, D), lambda i: (0, 0)),
        ],
        out_specs=[
            pl.BlockSpec((bm, D), lambda i: (i, 0)),
            pl.BlockSpec((bm, D), lambda i: (i, 0)),
        ],
        out_shape=[
            jax.ShapeDtypeStruct((n_pad, D), jnp.float32),
            jax.ShapeDtypeStruct((n_pad, D), jnp.float32),
        ],
    )(feat_pad, wt, wd)


def _make_sc_kernel(npw, nb):
    mesh = plsc.VectorSubcoreMesh(core_axis_name="c", subcore_axis_name="s")

    @functools.partial(
        pl.kernel,
        out_type=jax.ShapeDtypeStruct((NW, npw, D), jnp.float32),
        mesh=mesh,
        scratch_types=[
            pltpu.VMEM((nb, B * K), jnp.int32),
            pltpu.VMEM((B * K, D), jnp.float32),
            pltpu.VMEM((npw, D), jnp.float32),
            pltpu.VMEM((npw, D), jnp.float32),
            pltpu.SemaphoreType.DMA,
        ],
    )
    def sc_kernel(hsrc_hbm, hdst_hbm, idx_hbm, out_hbm,
                  idx_v, rows_v, hdst_v, out_v, sem):
        wid = lax.axis_index("s") * NC + lax.axis_index("c")
        pltpu.sync_copy(idx_hbm.at[wid], idx_v)
        pltpu.sync_copy(hdst_hbm.at[wid], hdst_v)

        def batch(s, carry):
            pltpu.async_copy(hsrc_hbm.at[idx_v.at[s]], rows_v, sem).wait()
            for b in range(B):
                base = b * K
                for c in range(D // L):
                    sl = pl.ds(c * L, L)
                    acc = rows_v[base, sl]
                    for j in range(1, K):
                        acc = jnp.maximum(acc, rows_v[base + j, sl])
                    out_v[s * B + b, sl] = acc + hdst_v[s * B + b, sl]
            return carry

        lax.fori_loop(0, nb, batch, 0)
        pltpu.sync_copy(out_v, out_hbm.at[wid])

    return sc_kernel


def kernel(k, src_ind, feat, W_theta, W_phi):
    n = feat.shape[0]
    n_pad = -(-n // 512) * 512
    npw = n_pad // NW
    nb = npw // B

    feat_pad = jnp.pad(feat, ((0, n_pad - n), (0, 0)))
    wt = W_theta.T
    wd = (W_phi - W_theta).T
    h_src, h_dst = _tc_matmuls(feat_pad, wt, wd, n_pad)

    idx = jnp.pad(src_ind.astype(jnp.int32), ((0, n_pad - n), (0, 0)))
    idx_r = idx.reshape(NW, nb, B * K)
    hdst_r = h_dst.reshape(NW, npw, D)

    out = _make_sc_kernel(npw, nb)(h_src, hdst_r, idx_r)
    return out.reshape(n_pad, D)[:n]


# speedup vs baseline: 1.7711x; 1.1515x over previous
import functools

import jax
import jax.numpy as jnp
from jax import lax
from jax.experimental import pallas as pl
from jax.experimental.pallas import tpu as pltpu
from jax.experimental.pallas import tpu_sc as plsc

D = 128
K = 32
NC = 2
NS = 16
NW = NC * NS
L = 16
B = 4


def _matmul_body(x_ref, wt_ref, wd_ref, hs_ref, hd_ref):
    x = x_ref[...]
    hs_ref[...] = jnp.dot(x, wt_ref[...], preferred_element_type=jnp.float32)
    hd_ref[...] = jnp.dot(x, wd_ref[...], preferred_element_type=jnp.float32)


def _tc_matmuls(feat_pad, wt, wd, n_pad):
    bm = 512
    grid = (n_pad // bm,)
    return pl.pallas_call(
        _matmul_body,
        grid=grid,
        in_specs=[
            pl.BlockSpec((bm, D), lambda i: (i, 0)),
            pl.BlockSpec((D, D), lambda i: (0, 0)),
            pl.BlockSpec((D, D), lambda i: (0, 0)),
        ],
        out_specs=[
            pl.BlockSpec((bm, D), lambda i: (i, 0)),
            pl.BlockSpec((bm, D), lambda i: (i, 0)),
        ],
        out_shape=[
            jax.ShapeDtypeStruct((n_pad, D), jnp.float32),
            jax.ShapeDtypeStruct((n_pad, D), jnp.float32),
        ],
    )(feat_pad, wt, wd)


def _make_sc_kernel(npw, nb):
    mesh = plsc.VectorSubcoreMesh(core_axis_name="c", subcore_axis_name="s")

    @functools.partial(
        pl.kernel,
        out_type=jax.ShapeDtypeStruct((NW, npw, D), jnp.float32),
        mesh=mesh,
        scratch_types=[
            pltpu.VMEM((nb, B * K), jnp.int32),
            pltpu.VMEM((2, B * K, D), jnp.float32),
            pltpu.VMEM((npw, D), jnp.float32),
            pltpu.VMEM((npw, D), jnp.float32),
            pltpu.SemaphoreType.DMA,
            pltpu.SemaphoreType.DMA,
        ],
    )
    def sc_kernel(hsrc_hbm, hdst_hbm, idx_hbm, out_hbm,
                  idx_v, rows_v, hdst_v, out_v, sem0, sem1):
        wid = lax.axis_index("s") * NC + lax.axis_index("c")
        pltpu.sync_copy(idx_hbm.at[wid], idx_v)
        pltpu.sync_copy(hdst_hbm.at[wid], hdst_v)
        sems = (sem0, sem1)

        def start(s, buf):
            pltpu.async_copy(hsrc_hbm.at[idx_v.at[s]], rows_v.at[buf],
                             sems[buf])

        def wait(s, buf):
            pltpu.make_async_copy(hsrc_hbm.at[idx_v.at[s]], rows_v.at[buf],
                                  sems[buf]).wait()

        def compute(s, buf):
            for b in range(B):
                base = b * K
                for c in range(D // L):
                    sl = pl.ds(c * L, L)
                    acc = rows_v[buf, base, sl]
                    for j in range(1, K):
                        acc = jnp.maximum(acc, rows_v[buf, base + j, sl])
                    out_v[s * B + b, sl] = acc + hdst_v[s * B + b, sl]

        start(0, 0)

        def pair(t, carry):
            s0 = 2 * t
            start(s0 + 1, 1)
            wait(s0, 0)
            compute(s0, 0)

            @pl.when(s0 + 2 < nb)
            def _():
                start(s0 + 2, 0)

            wait(s0 + 1, 1)
            compute(s0 + 1, 1)
            return carry

        lax.fori_loop(0, nb // 2, pair, 0)
        pltpu.sync_copy(out_v, out_hbm.at[wid])

    return sc_kernel


def kernel(k, src_ind, feat, W_theta, W_phi):
    n = feat.shape[0]
    n_pad = -(-n // 512) * 512
    npw = n_pad // NW
    nb = npw // B

    feat_pad = jnp.pad(feat, ((0, n_pad - n), (0, 0)))
    wt = W_theta.T
    wd = (W_phi - W_theta).T
    h_src, h_dst = _tc_matmuls(feat_pad, wt, wd, n_pad)

    idx = jnp.pad(src_ind.astype(jnp.int32), ((0, n_pad - n), (0, 0)))
    idx_r = idx.reshape(NW, nb, B * K)
    hdst_r = h_dst.reshape(NW, npw, D)

    out = _make_sc_kernel(npw, nb)(h_src, hdst_r, idx_r)
    return out.reshape(n_pad, D)[:n]


# speedup vs baseline: 2.1553x; 1.2169x over previous
import functools

import jax
import jax.numpy as jnp
from jax import lax
from jax.experimental import pallas as pl
from jax.experimental.pallas import tpu as pltpu
from jax.experimental.pallas import tpu_sc as plsc

D = 128
D2 = D // 2
K = 32
NC = 2
NS = 16
NW = NC * NS
L = 16
B = 4


def _matmul_body(x_ref, wt_ref, wd_ref, hs_ref, hd_ref):
    x = x_ref[...]
    hs = jnp.dot(x, wt_ref[...], preferred_element_type=jnp.float32)
    hd = jnp.dot(x, wd_ref[...], preferred_element_type=jnp.float32)
    hs_ref[...] = hs.astype(jnp.bfloat16)
    hd_ref[...] = hd.astype(jnp.bfloat16)


def _tc_matmuls(feat_pad, wt, wd, n_pad):
    bm = 512
    grid = (n_pad // bm,)
    return pl.pallas_call(
        _matmul_body,
        grid=grid,
        in_specs=[
            pl.BlockSpec((bm, D), lambda i: (i, 0)),
            pl.BlockSpec((D, D), lambda i: (0, 0)),
            pl.BlockSpec((D, D), lambda i: (0, 0)),
        ],
        out_specs=[
            pl.BlockSpec((bm, D), lambda i: (i, 0)),
            pl.BlockSpec((bm, D), lambda i: (i, 0)),
        ],
        out_shape=[
            jax.ShapeDtypeStruct((n_pad, D), jnp.bfloat16),
            jax.ShapeDtypeStruct((n_pad, D), jnp.bfloat16),
        ],
    )(feat_pad, wt, wd)


def _make_sc_kernel(npw, nb):
    mesh = plsc.VectorSubcoreMesh(core_axis_name="c", subcore_axis_name="s")

    @functools.partial(
        pl.kernel,
        out_type=jax.ShapeDtypeStruct((NW, npw, D2), jnp.int32),
        mesh=mesh,
        compiler_params=pltpu.CompilerParams(use_tc_tiling_on_sc=False, needs_layout_passes=False),
        scratch_types=[
            pltpu.VMEM((nb, B * K), jnp.int32),
            pltpu.VMEM((2, B * K, D2), jnp.int32),
            pltpu.VMEM((2 * B, D2), jnp.int32),
            pltpu.VMEM((2 * B, D2), jnp.int32),
            pltpu.SemaphoreType.DMA,
            pltpu.SemaphoreType.DMA,
        ],
    )
    def sc_kernel(hsrc_hbm, hdst_hbm, idx_hbm, out_hbm,
                  idx_v, rows_v, hdst_v, out_v, sem0, sem1):
        wid = lax.axis_index("s") * NC + lax.axis_index("c")
        pltpu.sync_copy(idx_hbm.at[wid], idx_v)
        sems = (sem0, sem1)

        def start(s, buf):
            pltpu.async_copy(hsrc_hbm.at[idx_v.at[s]], rows_v.at[buf],
                             sems[buf])

        def wait(s, buf):
            pltpu.make_async_copy(hsrc_hbm.at[idx_v.at[s]], rows_v.at[buf],
                                  sems[buf]).wait()

        def compute(buf, half):
            for b in range(B):
                base = b * K
                for c in range(D2 // L):
                    sl = pl.ds(c * L, L)
                    acc = plsc.bitcast(rows_v[buf, base, sl], jnp.bfloat16)
                    for j in range(1, K):
                        x = plsc.bitcast(rows_v[buf, base + j, sl],
                                         jnp.bfloat16)
                        acc = jnp.maximum(acc, x)
                    r = half * B + b
                    hd = plsc.bitcast(hdst_v[r, sl], jnp.bfloat16)
                    out_v[r, sl] = plsc.bitcast(acc + hd, jnp.int32)

        start(0, 0)

        def pair(t, carry):
            s0 = 2 * t
            start(s0 + 1, 1)
            pltpu.sync_copy(hdst_hbm.at[wid, pl.ds(t * 2 * B, 2 * B)], hdst_v)
            wait(s0, 0)
            compute(0, 0)

            @pl.when(s0 + 2 < nb)
            def _():
                start(s0 + 2, 0)

            wait(s0 + 1, 1)
            compute(1, 1)
            pltpu.sync_copy(out_v, out_hbm.at[wid, pl.ds(t * 2 * B, 2 * B)])
            return carry

        lax.fori_loop(0, nb // 2, pair, 0)

    return sc_kernel


def kernel(k, src_ind, feat, W_theta, W_phi):
    n = feat.shape[0]
    n_pad = -(-n // 512) * 512
    npw = n_pad // NW
    nb = npw // B

    feat_pad = jnp.pad(feat, ((0, n_pad - n), (0, 0)))
    wt = W_theta.T
    wd = (W_phi - W_theta).T
    h_src, h_dst = _tc_matmuls(feat_pad, wt, wd, n_pad)

    idx = jnp.pad(src_ind.astype(jnp.int32), ((0, n_pad - n), (0, 0)))
    idx_r = idx.reshape(NW, nb, B * K)

    def pack_i32(x):
        return lax.bitcast_convert_type(
            x.reshape(x.shape[0], D2, 2), jnp.int32)

    hsrc_p = pack_i32(h_src)
    hdst_p = pack_i32(h_dst).reshape(NW, npw, D2)

    out = _make_sc_kernel(npw, nb)(hsrc_p, hdst_p, idx_r)
    out_bf = lax.bitcast_convert_type(out, jnp.bfloat16)
    return out_bf.reshape(n_pad, D)[:n].astype(jnp.float32)


# speedup vs baseline: 4.3425x; 2.0148x over previous
import functools

import jax
import jax.numpy as jnp
from jax import lax
from jax.experimental import pallas as pl
from jax.experimental.pallas import tpu as pltpu
from jax.experimental.pallas import tpu_sc as plsc

D = 128
D2 = D // 2
K = 32
NC = 2
NS = 16
NW = NC * NS
CPW = D2 // NW
L = 16
CH = 1024


def _matmul_body(x_ref, wt_ref, wd_ref, hs_ref, hd_ref):
    x = x_ref[...]
    hs = jnp.dot(x, wt_ref[...], preferred_element_type=jnp.float32)
    hd = jnp.dot(x, wd_ref[...], preferred_element_type=jnp.float32)
    hs_ref[...] = hs.astype(jnp.bfloat16)
    hd_ref[...] = hd.astype(jnp.bfloat16)


def _tc_matmuls(feat_pad, wt, wd, n_pad):
    bm = 512
    grid = (n_pad // bm,)
    return pl.pallas_call(
        _matmul_body,
        grid=grid,
        in_specs=[
            pl.BlockSpec((bm, D), lambda i: (i, 0)),
            pl.BlockSpec((D, D), lambda i: (0, 0)),
            pl.BlockSpec((D, D), lambda i: (0, 0)),
        ],
        out_specs=[
            pl.BlockSpec((bm, D), lambda i: (i, 0)),
            pl.BlockSpec((bm, D), lambda i: (i, 0)),
        ],
        out_shape=[
            jax.ShapeDtypeStruct((n_pad, D), jnp.bfloat16),
            jax.ShapeDtypeStruct((n_pad, D), jnp.bfloat16),
        ],
    )(feat_pad, wt, wd)


def _make_sc_kernel(n_pad):
    nch = n_pad // CH
    mesh = plsc.VectorSubcoreMesh(core_axis_name="c", subcore_axis_name="s")

    @functools.partial(
        pl.kernel,
        out_type=jax.ShapeDtypeStruct((D2, n_pad), jnp.int32),
        mesh=mesh,
        compiler_params=pltpu.CompilerParams(
            use_tc_tiling_on_sc=False, needs_layout_passes=False),
        scratch_types=[
            pltpu.VMEM((CPW, n_pad), jnp.int32),
            pltpu.VMEM((CPW, n_pad), jnp.int32),
            pltpu.VMEM((2, K, CH), jnp.int32),
            pltpu.VMEM((2, CPW, CH), jnp.int32),
            pltpu.SemaphoreType.DMA,
            pltpu.SemaphoreType.DMA,
            pltpu.SemaphoreType.DMA,
            pltpu.SemaphoreType.DMA,
        ],
    )
    def sc_kernel(hsrc_hbm, hdst_hbm, idx_hbm, out_hbm,
                  tab_v, hdst_v, idx_v, out_v, isem0, isem1, osem0, osem1):
        wid = lax.axis_index("s") * NC + lax.axis_index("c")
        col0 = wid * CPW
        pltpu.sync_copy(hsrc_hbm.at[pl.ds(col0, CPW)], tab_v)
        pltpu.sync_copy(hdst_hbm.at[pl.ds(col0, CPW)], hdst_v)
        isems = (isem0, isem1)
        osems = (osem0, osem1)

        def idx_start(ch, buf):
            pltpu.async_copy(idx_hbm.at[:, pl.ds(ch * CH, CH)],
                             idx_v.at[buf], isems[buf])

        def idx_wait(ch, buf):
            pltpu.make_async_copy(idx_hbm.at[:, pl.ds(ch * CH, CH)],
                                  idx_v.at[buf], isems[buf]).wait()

        def out_start(ch, buf):
            pltpu.async_copy(
                out_v.at[buf],
                out_hbm.at[pl.ds(col0, CPW), pl.ds(ch * CH, CH)],
                osems[buf])

        def out_wait(ch, buf):
            pltpu.make_async_copy(
                out_v.at[buf],
                out_hbm.at[pl.ds(col0, CPW), pl.ds(ch * CH, CH)],
                osems[buf]).wait()

        def chunk_compute(buf):
            def group(g, carry):
                gsl = pl.ds(g * L, L)
                accs = []
                for col in range(CPW):
                    iv = idx_v[buf, 0, gsl]
                    x = plsc.load_gather(tab_v.at[col], [iv])
                    accs.append(plsc.bitcast(x, jnp.bfloat16))
                for j in range(1, K):
                    iv = idx_v[buf, j, gsl]
                    for col in range(CPW):
                        x = plsc.load_gather(tab_v.at[col], [iv])
                        accs[col] = jnp.maximum(
                            accs[col], plsc.bitcast(x, jnp.bfloat16))
                for col in range(CPW):
                    hd = plsc.bitcast(hdst_v[col, gsl], jnp.bfloat16)
                    out_v[buf, col, gsl] = plsc.bitcast(
                        accs[col] + hd, jnp.int32)
                return carry

            lax.fori_loop(0, CH // L, group, 0)

        idx_start(0, 0)
        for ch in range(nch):
            buf = ch % 2
            if ch + 1 < nch:
                idx_start(ch + 1, 1 - buf)
            idx_wait(ch, buf)
            if ch >= 2:
                out_wait(ch - 2, buf)
            chunk_compute(buf)
            out_start(ch, buf)
        out_wait(nch - 2, nch % 2)
        out_wait(nch - 1, (nch - 1) % 2)

    return sc_kernel


def kernel(k, src_ind, feat, W_theta, W_phi):
    n = feat.shape[0]
    n_pad = -(-n // CH) * CH
    feat_pad = jnp.pad(feat, ((0, n_pad - n), (0, 0)))
    wt = W_theta.T
    wd = (W_phi - W_theta).T
    h_src, h_dst = _tc_matmuls(feat_pad, wt, wd, n_pad)

    idx_t = jnp.pad(src_ind.astype(jnp.int32),
                    ((0, n_pad - n), (0, 0))).T

    def pack_t(x):
        return lax.bitcast_convert_type(
            x.reshape(n_pad, D2, 2), jnp.int32).T

    out = _make_sc_kernel(n_pad)(pack_t(h_src), pack_t(h_dst), idx_t)
    out_bf = lax.bitcast_convert_type(out, jnp.bfloat16)
    out_f = out_bf.transpose(1, 0, 2).reshape(n_pad, D)
    return out_f[:n].astype(jnp.float32)
